# baseline (speedup 1.0000x reference)
import functools

import jax
import jax.numpy as jnp
from jax import lax
from jax.experimental import pallas as pl
from jax.experimental.pallas import tpu as pltpu
from jax.experimental.pallas import tpu_sc as plsc

N = 10000
E = 320000
D = 128
EPS = 1e-5

NC = 2
NS = 16
NW = NC * NS
C = 128
K = 80
EPT = K * C
E_pad = EPT * NW
GRP = 8
NG = K // GRP
K0 = 104
K1 = 56
E0 = NS * K0 * C
E1 = NS * K1 * C
N_pad = 10112
RPT = N_pad // NS
DEG_W = 128


def _sc_agg_body(h_ref, src_ref, dst_ref, zf_ref, acc_out,
                 acc_sh, sg, dst_v, rows0, rows1, g0s, g1s, s0s, s1s, isem):
    c = lax.axis_index("c")
    s = lax.axis_index("s")
    wid = c * NS + s

    pltpu.sync_copy(zf_ref, acc_sh.at[pl.ds(s * RPT, RPT)])
    pltpu.sync_copy(dst_ref.at[wid], dst_v)
    pltpu.sync_copy(src_ref.at[wid, pl.ds(0, GRP)], sg.at[pl.ds(0, GRP)])
    plsc.subcore_barrier()

    def serial_loop(kk):
        pend = None
        for g in range(kk // GRP):
            if g + 1 < kk // GRP:
                pend = pltpu.async_copy(
                    src_ref.at[wid, pl.ds((g + 1) * GRP, GRP)],
                    sg.at[pl.ds(((g + 1) % 2) * GRP, GRP)], isem)
            cur = (g % 2) * GRP

            def chunk(ii, carry):
                j0 = g * GRP + ii
                pltpu.async_copy(h_ref.at[sg.at[cur + ii]], rows0, g0s).wait()
                pltpu.async_copy(rows0, acc_sh.at[dst_v.at[j0]], s0s,
                                 add=True).wait()
                return carry

            lax.fori_loop(0, GRP, chunk, 0)
            if g + 1 < kk // GRP:
                pend.wait()

    @pl.when(c == 0)
    def _core0():
        serial_loop(K0)

    @pl.when(c == 1)
    def _core1():
        serial_loop(K1)

    plsc.subcore_barrier()

    pltpu.sync_copy(acc_sh.at[pl.ds(s * RPT, RPT)],
                    acc_out.at[c, pl.ds(s * RPT, RPT)])


def _make_sc_agg():
    mesh = plsc.VectorSubcoreMesh(core_axis_name="c", subcore_axis_name="s")
    return pl.kernel(
        _sc_agg_body,
        out_type=[jax.ShapeDtypeStruct((NC, N_pad, D), jnp.float32)],
        mesh=mesh,
        scratch_types=[
            pltpu.VMEM_SHARED((N_pad, D), jnp.float32),
            pltpu.VMEM((2 * GRP, C), jnp.int32),
            pltpu.VMEM((K0, C), jnp.int32),
            pltpu.VMEM((C, D), jnp.float32),
            pltpu.VMEM((C, D), jnp.float32),
            pltpu.SemaphoreType.DMA,
            pltpu.SemaphoreType.DMA,
            pltpu.SemaphoreType.DMA,
            pltpu.SemaphoreType.DMA,
            pltpu.SemaphoreType.DMA,
        ],
    )


def _sc_deg_body(dst_ref, zd_ref, ones_ref, deg_out,
                 deg_sh, dst_v, ones_v, sem):
    c = lax.axis_index("c")
    s = lax.axis_index("s")
    wid = c * NS + s

    pltpu.sync_copy(zd_ref, deg_sh.at[pl.ds(s * RPT, RPT)])
    pltpu.sync_copy(ones_ref, ones_v)
    pltpu.sync_copy(dst_ref.at[wid], dst_v)
    plsc.subcore_barrier()

    def chunk(j, carry):
        pltpu.async_copy(ones_v, deg_sh.at[dst_v.at[j]], sem, add=True).wait()
        return carry

    lax.fori_loop(0, K, chunk, 0)
    plsc.subcore_barrier()
    pltpu.sync_copy(deg_sh.at[pl.ds(s * RPT, RPT)],
                    deg_out.at[c, pl.ds(s * RPT, RPT)])


def _make_sc_deg():
    mesh = plsc.VectorSubcoreMesh(core_axis_name="c", subcore_axis_name="s")
    return pl.kernel(
        _sc_deg_body,
        out_type=[jax.ShapeDtypeStruct((NC, N_pad, DEG_W), jnp.float32)],
        mesh=mesh,
        scratch_types=[
            pltpu.VMEM_SHARED((N_pad, DEG_W), jnp.float32),
            pltpu.VMEM((K, C), jnp.int32),
            pltpu.VMEM((C, DEG_W), jnp.float32),
            pltpu.SemaphoreType.DMA,
        ],
    )


def _tc_layer_body(relu, h_ref, accA_ref, accB_ref, degA_ref, degB_ref,
                   wlT_ref, bl_ref, wrT_ref, g_ref, be_ref, out_ref):
    mask = lax.broadcasted_iota(jnp.int32, (N_pad, 1), 0) < N
    ssum = accA_ref[:] + accB_ref[:]
    deg = degA_ref[:, 0:1] + degB_ref[:, 0:1]
    agg = ssum / jnp.clip(deg, 1.0, None)
    xr = (jnp.dot(agg, wlT_ref[:], preferred_element_type=jnp.float32)
          + bl_ref[:]
          + jnp.dot(h_ref[:], wrT_ref[:], preferred_element_type=jnp.float32))
    xr_m = jnp.where(mask, xr, 0.0)
    mean = jnp.sum(xr_m, axis=0, keepdims=True) * (1.0 / N)
    d = jnp.where(mask, xr - mean, 0.0)
    var = jnp.sum(d * d, axis=0, keepdims=True) * (1.0 / N)
    y = (xr - mean) * (g_ref[:] * lax.rsqrt(var + EPS)) + be_ref[:]
    if relu:
        y = jnp.maximum(y, 0.0)
    out_ref[:] = jnp.where(mask, y, 0.0)


def _make_tc_layer(relu):
    return pl.pallas_call(
        functools.partial(_tc_layer_body, relu),
        out_shape=jax.ShapeDtypeStruct((N_pad, D), jnp.float32),
    )


@jax.jit
def kernel(x, edge_index, Wl1, bl1, Wr1, g1, be1, Wl2, bl2, Wr2, g2, be2,
           Wl3, bl3, Wr3, g3, be3):
    src = edge_index[0].astype(jnp.int32)
    dst = edge_index[1].astype(jnp.int32)
    pad = E_pad - E
    src_f = jnp.concatenate([src, jnp.full((pad,), N, jnp.int32)])
    dst_f = jnp.concatenate([dst, jnp.full((pad,), N, jnp.int32)])
    dst_p = dst_f.reshape(NW, K, C)

    def ragged(e):
        c0 = e[:E0].reshape(NS, K0, C)
        c1 = e[E0:].reshape(NS, K1, C)
        c1 = jnp.pad(c1, ((0, 0), (0, K0 - K1), (0, 0)), constant_values=N)
        return jnp.concatenate([c0, c1], axis=0)

    src_p = ragged(src_f)
    dst_r = ragged(dst_f)
    x_p = jnp.concatenate([x, jnp.zeros((N_pad - N, D), jnp.float32)], axis=0)

    zf = jnp.zeros((RPT, D), jnp.float32)
    zd = jnp.zeros((RPT, DEG_W), jnp.float32)
    ones_rows = jnp.ones((C, DEG_W), jnp.float32)

    sc_agg = _make_sc_agg()
    sc_deg = _make_sc_deg()

    (degp,) = sc_deg(dst_p, zd, ones_rows)
    degA, degB = degp[0], degp[1]
    (acc1,) = sc_agg(x_p, src_p, dst_r, zf)

    def layer(h, acc, Wl, bl, Wr, g, be, relu):
        return _make_tc_layer(relu)(
            h, acc[0], acc[1], degA, degB,
            Wl.T, bl.reshape(1, D), Wr.T, g.reshape(1, D), be.reshape(1, D))

    h1 = layer(x_p, acc1, Wl1, bl1, Wr1, g1, be1, True)
    (acc2,) = sc_agg(h1, src_p, dst_r, zf)
    h2 = layer(h1, acc2, Wl2, bl2, Wr2, g2, be2, True)
    (acc3,) = sc_agg(h2, src_p, dst_r, zf)
    out = layer(h2, acc3, Wl3, bl3, Wr3, g3, be3, False)
    return out[:N]


# speedup vs baseline: 1.0360x; 1.0360x over previous
import functools

import jax
import jax.numpy as jnp
from jax import lax
from jax.experimental import pallas as pl
from jax.experimental.pallas import tpu as pltpu
from jax.experimental.pallas import tpu_sc as plsc

N = 10000
E = 320000
D = 128
EPS = 1e-5

NC = 2
NS = 16
NW = NC * NS
C = 128
K = 80
EPT = K * C
E_pad = EPT * NW
GRP = 8
NG = K // GRP
N_pad = 10112
RPT = N_pad // NS
DEG_W = 128


def _sc_agg_body(h_ref, src_ref, dst_ref, zf_ref, acc_out,
                 acc_sh, sg, dst_v, rows0, g0s, s0s):
    c = lax.axis_index("c")
    s = lax.axis_index("s")
    wid = c * NS + s

    pltpu.sync_copy(zf_ref, acc_sh.at[pl.ds(s * RPT, RPT)])
    pltpu.sync_copy(src_ref.at[wid], sg)
    pltpu.sync_copy(dst_ref.at[wid], dst_v)
    plsc.subcore_barrier()

    def chunk(j, carry):
        pltpu.async_copy(h_ref.at[sg.at[j]], rows0, g0s).wait()
        pltpu.async_copy(rows0, acc_sh.at[dst_v.at[j]], s0s, add=True).wait()
        return carry

    lax.fori_loop(0, K, chunk, 0)

    plsc.subcore_barrier()

    pltpu.sync_copy(acc_sh.at[pl.ds(s * RPT, RPT)],
                    acc_out.at[c, pl.ds(s * RPT, RPT)])


def _make_sc_agg():
    mesh = plsc.VectorSubcoreMesh(core_axis_name="c", subcore_axis_name="s")
    return pl.kernel(
        _sc_agg_body,
        out_type=[jax.ShapeDtypeStruct((NC, N_pad, D), jnp.float32)],
        mesh=mesh,
        scratch_types=[
            pltpu.VMEM_SHARED((N_pad, D), jnp.float32),
            pltpu.VMEM((K, C), jnp.int32),
            pltpu.VMEM((K, C), jnp.int32),
            pltpu.VMEM((C, D), jnp.float32),
            pltpu.SemaphoreType.DMA,
            pltpu.SemaphoreType.DMA,
        ],
    )


def _sc_deg_body(dst_ref, zd_ref, ones_ref, deg_out,
                 deg_sh, dst_v, ones_v, sem):
    c = lax.axis_index("c")
    s = lax.axis_index("s")
    wid = c * NS + s

    pltpu.sync_copy(zd_ref, deg_sh.at[pl.ds(s * RPT, RPT)])
    pltpu.sync_copy(ones_ref, ones_v)
    pltpu.sync_copy(dst_ref.at[wid], dst_v)
    plsc.subcore_barrier()

    def chunk(j, carry):
        pltpu.async_copy(ones_v, deg_sh.at[dst_v.at[j]], sem, add=True).wait()
        return carry

    lax.fori_loop(0, K, chunk, 0)
    plsc.subcore_barrier()
    pltpu.sync_copy(deg_sh.at[pl.ds(s * RPT, RPT)],
                    deg_out.at[c, pl.ds(s * RPT, RPT)])


def _make_sc_deg():
    mesh = plsc.VectorSubcoreMesh(core_axis_name="c", subcore_axis_name="s")
    return pl.kernel(
        _sc_deg_body,
        out_type=[jax.ShapeDtypeStruct((NC, N_pad, DEG_W), jnp.float32)],
        mesh=mesh,
        scratch_types=[
            pltpu.VMEM_SHARED((N_pad, DEG_W), jnp.float32),
            pltpu.VMEM((K, C), jnp.int32),
            pltpu.VMEM((C, DEG_W), jnp.float32),
            pltpu.SemaphoreType.DMA,
        ],
    )


def _tc_layer_body(relu, h_ref, accA_ref, accB_ref, degA_ref, degB_ref,
                   wlT_ref, bl_ref, wrT_ref, g_ref, be_ref, out_ref):
    mask = lax.broadcasted_iota(jnp.int32, (N_pad, 1), 0) < N
    ssum = accA_ref[:] + accB_ref[:]
    deg = degA_ref[:, 0:1] + degB_ref[:, 0:1]
    agg = ssum / jnp.clip(deg, 1.0, None)
    xr = (jnp.dot(agg, wlT_ref[:], preferred_element_type=jnp.float32)
          + bl_ref[:]
          + jnp.dot(h_ref[:], wrT_ref[:], preferred_element_type=jnp.float32))
    xr_m = jnp.where(mask, xr, 0.0)
    mean = jnp.sum(xr_m, axis=0, keepdims=True) * (1.0 / N)
    d = jnp.where(mask, xr - mean, 0.0)
    var = jnp.sum(d * d, axis=0, keepdims=True) * (1.0 / N)
    y = (xr - mean) * (g_ref[:] * lax.rsqrt(var + EPS)) + be_ref[:]
    if relu:
        y = jnp.maximum(y, 0.0)
    out_ref[:] = jnp.where(mask, y, 0.0)


def _make_tc_layer(relu):
    return pl.pallas_call(
        functools.partial(_tc_layer_body, relu),
        out_shape=jax.ShapeDtypeStruct((N_pad, D), jnp.float32),
    )


@jax.jit
def kernel(x, edge_index, Wl1, bl1, Wr1, g1, be1, Wl2, bl2, Wr2, g2, be2,
           Wl3, bl3, Wr3, g3, be3):
    src = edge_index[0].astype(jnp.int32)
    dst = edge_index[1].astype(jnp.int32)
    pad = E_pad - E
    src_p = jnp.concatenate([src, jnp.full((pad,), N, jnp.int32)]).reshape(NW, K, C)
    dst_p = jnp.concatenate([dst, jnp.full((pad,), N, jnp.int32)]).reshape(NW, K, C)
    x_p = jnp.concatenate([x, jnp.zeros((N_pad - N, D), jnp.float32)], axis=0)

    zf = jnp.zeros((RPT, D), jnp.float32)
    zd = jnp.zeros((RPT, DEG_W), jnp.float32)
    ones_rows = jnp.ones((C, DEG_W), jnp.float32)

    sc_agg = _make_sc_agg()
    sc_deg = _make_sc_deg()

    (degp,) = sc_deg(dst_p, zd, ones_rows)
    degA, degB = degp[0], degp[1]
    (acc1,) = sc_agg(x_p, src_p, dst_p, zf)

    def layer(h, acc, Wl, bl, Wr, g, be, relu):
        return _make_tc_layer(relu)(
            h, acc[0], acc[1], degA, degB,
            Wl.T, bl.reshape(1, D), Wr.T, g.reshape(1, D), be.reshape(1, D))

    h1 = layer(x_p, acc1, Wl1, bl1, Wr1, g1, be1, True)
    (acc2,) = sc_agg(h1, src_p, dst_p, zf)
    h2 = layer(h1, acc2, Wl2, bl2, Wr2, g2, be2, True)
    (acc3,) = sc_agg(h2, src_p, dst_p, zf)
    out = layer(h2, acc3, Wl3, bl3, Wr3, g3, be3, False)
    return out[:N]


# speedup vs baseline: 1.5680x; 1.5136x over previous
import functools

import jax
import jax.numpy as jnp
from jax import lax
from jax.experimental import pallas as pl
from jax.experimental.pallas import tpu as pltpu
from jax.experimental.pallas import tpu_sc as plsc

N = 10000
E = 320000
D = 128
EPS = 1e-5

NC = 2
NS = 16
NW = NC * NS
C = 128
K = -(-(E // NW) // C)
EPT = K * C
E_pad = EPT * NW
N_pad = 10112
RPT = N_pad // NS
DEG_W = 128


def _sc_agg_body(h_ref, src_ref, dst_ref, zf_ref, acc_out,
                 acc_sh, src_v, dst_v, rows_v, sem):
    c = lax.axis_index("c")
    s = lax.axis_index("s")
    wid = c * NS + s

    pltpu.sync_copy(zf_ref, acc_sh.at[pl.ds(s * RPT, RPT)])
    pltpu.sync_copy(src_ref.at[wid], src_v)
    pltpu.sync_copy(dst_ref.at[wid], dst_v)
    plsc.subcore_barrier()

    def chunk(j, carry):
        pltpu.async_copy(h_ref.at[src_v.at[j]], rows_v, sem).wait()
        pltpu.async_copy(rows_v, acc_sh.at[dst_v.at[j]], sem, add=True).wait()
        return carry

    lax.fori_loop(0, K, chunk, 0)

    plsc.subcore_barrier()

    pltpu.sync_copy(acc_sh.at[pl.ds(s * RPT, RPT)],
                    acc_out.at[c, pl.ds(s * RPT, RPT)])


def _make_sc_agg():
    mesh = plsc.VectorSubcoreMesh(core_axis_name="c", subcore_axis_name="s")
    return pl.kernel(
        _sc_agg_body,
        out_type=[jax.ShapeDtypeStruct((NC, N_pad, D), jnp.float32)],
        mesh=mesh,
        scratch_types=[
            pltpu.VMEM_SHARED((N_pad, D), jnp.float32),
            pltpu.VMEM((K, C), jnp.int32),
            pltpu.VMEM((K, C), jnp.int32),
            pltpu.VMEM((C, D), jnp.float32),
            pltpu.SemaphoreType.DMA,
        ],
    )


def _sc_deg_body(dst_ref, zd_ref, ones_ref, deg_out,
                 deg_sh, dst_v, ones_v, sem):
    c = lax.axis_index("c")
    s = lax.axis_index("s")
    wid = c * NS + s

    pltpu.sync_copy(zd_ref, deg_sh.at[pl.ds(s * RPT, RPT)])
    pltpu.sync_copy(ones_ref, ones_v)
    pltpu.sync_copy(dst_ref.at[wid], dst_v)
    plsc.subcore_barrier()

    def chunk(j, carry):
        pltpu.async_copy(ones_v, deg_sh.at[dst_v.at[j]], sem, add=True).wait()
        return carry

    lax.fori_loop(0, K, chunk, 0)
    plsc.subcore_barrier()
    pltpu.sync_copy(deg_sh.at[pl.ds(s * RPT, RPT)],
                    deg_out.at[c, pl.ds(s * RPT, RPT)])


def _make_sc_deg():
    mesh = plsc.VectorSubcoreMesh(core_axis_name="c", subcore_axis_name="s")
    return pl.kernel(
        _sc_deg_body,
        out_type=[jax.ShapeDtypeStruct((NC, N_pad, DEG_W), jnp.float32)],
        mesh=mesh,
        scratch_types=[
            pltpu.VMEM_SHARED((N_pad, DEG_W), jnp.float32),
            pltpu.VMEM((K, C), jnp.int32),
            pltpu.VMEM((C, DEG_W), jnp.float32),
            pltpu.SemaphoreType.DMA,
        ],
    )


def _tc_layer_body(relu, h_ref, accA_ref, accB_ref, degA_ref, degB_ref,
                   wlT_ref, bl_ref, wrT_ref, g_ref, be_ref, out_ref):
    mask = lax.broadcasted_iota(jnp.int32, (N_pad, 1), 0) < N
    ssum = accA_ref[:] + accB_ref[:]
    deg = degA_ref[:, 0:1] + degB_ref[:, 0:1]
    agg = ssum / jnp.clip(deg, 1.0, None)
    xr = (jnp.dot(agg, wlT_ref[:], preferred_element_type=jnp.float32)
          + bl_ref[:]
          + jnp.dot(h_ref[:], wrT_ref[:], preferred_element_type=jnp.float32))
    xr_m = jnp.where(mask, xr, 0.0)
    mean = jnp.sum(xr_m, axis=0, keepdims=True) * (1.0 / N)
    d = jnp.where(mask, xr - mean, 0.0)
    var = jnp.sum(d * d, axis=0, keepdims=True) * (1.0 / N)
    y = (xr - mean) * (g_ref[:] * lax.rsqrt(var + EPS)) + be_ref[:]
    if relu:
        y = jnp.maximum(y, 0.0)
    out_ref[:] = jnp.where(mask, y, 0.0)


def _make_tc_layer(relu):
    return pl.pallas_call(
        functools.partial(_tc_layer_body, relu),
        out_shape=jax.ShapeDtypeStruct((N_pad, D), jnp.float32),
    )


@jax.jit
def kernel(x, edge_index, Wl1, bl1, Wr1, g1, be1, Wl2, bl2, Wr2, g2, be2,
           Wl3, bl3, Wr3, g3, be3):
    src = edge_index[0].astype(jnp.int32)
    dst = edge_index[1].astype(jnp.int32)
    pad = E_pad - E
    src_p = jnp.concatenate([src, jnp.full((pad,), N, jnp.int32)]).reshape(NW, K, C)
    dst_p = jnp.concatenate([dst, jnp.full((pad,), N, jnp.int32)]).reshape(NW, K, C)
    x_p = jnp.concatenate([x, jnp.zeros((N_pad - N, D), jnp.float32)], axis=0)

    zf = jnp.zeros((RPT, D), jnp.float32)
    zd = jnp.zeros((RPT, DEG_W), jnp.float32)
    ones_rows = jnp.ones((C, DEG_W), jnp.float32)

    sc_agg = _make_sc_agg()
    sc_deg = _make_sc_deg()

    (degp,) = sc_deg(dst_p, zd, ones_rows)
    degA, degB = degp[0], degp[1]
    (acc1,) = sc_agg(x_p, src_p, dst_p, zf)

    def layer(h, acc, Wl, bl, Wr, g, be, relu):
        return _make_tc_layer(relu)(
            h, acc[0], acc[1], degA, degB,
            Wl.T, bl.reshape(1, D), Wr.T, g.reshape(1, D), be.reshape(1, D))

    h1 = layer(x_p, acc1, Wl1, bl1, Wr1, g1, be1, True)
    (acc2,) = sc_agg(h1, src_p, dst_p, zf)
    h2 = layer(h1, acc2, Wl2, bl2, Wr2, g2, be2, True)
    (acc3,) = sc_agg(h2, src_p, dst_p, zf)
    out = layer(h2, acc3, Wl3, bl3, Wr3, g3, be3, False)
    return out[:N]
